# baseline (speedup 1.0000x reference)
import math

import jax
import jax.numpy as jnp
from jax.experimental import pallas as pl
from jax.experimental.pallas import tpu as pltpu
from jax.experimental.pallas import tpu_sc as plsc

TK = 8
SC_LANES = 16
SC_SUBCORES = 16
SC_CORES = 2


def _sc_xstd_body(a_hbm, x_hbm, eps_hbm, o_hbm, a_loc, x_loc, eps_loc,
                  o_loc, sem):
    n = x_loc.shape[0]
    d = x_loc.shape[1]
    rows = a_loc.shape[0]
    core = jax.lax.axis_index("core")
    sub = jax.lax.axis_index("subcore")
    wid = core * SC_SUBCORES + sub
    r0 = wid * rows
    b = r0 // n
    i0 = r0 - b * n
    pltpu.async_copy(eps_hbm, eps_loc, sem).wait()
    pltpu.async_copy(a_hbm.at[pl.ds(r0, rows), :], a_loc, sem).wait()
    pltpu.async_copy(x_hbm.at[pl.ds(b * n, n), :], x_loc, sem).wait()
    s_vec = eps_loc[pl.ds(0, SC_LANES)] + 1.0
    nc = d // SC_LANES

    @pl.loop(0, rows)
    def _row(li):
        for c in range(nc):
            sl = pl.ds(c * SC_LANES, SC_LANES)
            o_loc[li, sl] = s_vec * x_loc[i0 + li, sl]

        @pl.loop(0, n // SC_LANES)
        def _chunk(jc):
            av = a_loc[li, pl.ds(jc * SC_LANES, SC_LANES)]
            for ll in range(SC_LANES):
                j = jc * SC_LANES + ll

                @pl.when(av[ll] != 0.0)
                def _(j=j):
                    for c in range(nc):
                        sl = pl.ds(c * SC_LANES, SC_LANES)
                        o_loc[li, sl] = o_loc[li, sl] + x_loc[j, sl]

    pltpu.async_copy(o_loc, o_hbm.at[pl.ds(r0, rows), :], sem).wait()


def _sc_xstd(a2, x2, eps16):
    bn, n = x2.shape[0], x2.shape[1]
    rows = bn // (SC_CORES * SC_SUBCORES)
    mesh = plsc.VectorSubcoreMesh(core_axis_name="core",
                                  subcore_axis_name="subcore")
    kern = pl.kernel(
        _sc_xstd_body,
        out_type=jax.ShapeDtypeStruct((bn, n), jnp.float32),
        mesh=mesh,
        scratch_types=[
            pltpu.VMEM((rows, n), jnp.float32),
            pltpu.VMEM((n, n), jnp.float32),
            pltpu.VMEM((SC_LANES,), jnp.float32),
            pltpu.VMEM((rows, n), jnp.float32),
            pltpu.SemaphoreType.DMA,
        ],
    )
    return kern(a2, x2, eps16)


def _pairs_body(a_ref, x_ref, v1_ref, c1_ref, v2_ref, c2_ref,
                out_ref, q_ref, g_ref):
    n = a_ref.shape[1]
    d = x_ref.shape[2]
    nt = n // TK
    f32 = jnp.float32
    a = a_ref[0]
    x = x_ref[0]
    g_ref[:, :] = jnp.dot(x, v1_ref[:, :], preferred_element_type=f32) \
        + 0.5 * c1_ref[:, :]
    G = g_ref[:, :]
    c2 = c2_ref[:, :]
    v2 = v2_ref[:, :]
    rowi = jax.lax.broadcasted_iota(jnp.int32, (n, n), 0)
    coli = jax.lax.broadcasted_iota(jnp.int32, (n, n), 1)
    at = jnp.where(rowi < coli, a, 0.0)
    for t in range(nt):
        rmax = (t + 1) * TK
        gk = G[t * TK:(t + 1) * TK, :]
        h = gk[:, None, :] + G[:rmax][None, :, :]
        h = jnp.maximum(h, 0.0).reshape(TK * rmax, -1)
        p = jnp.dot(h, v2, preferred_element_type=f32) + c2
        p = jnp.maximum(p, 0.0)
        for kk in range(TK):
            k = t * TK + kk
            q_ref[:rmax, k * d:(k + 1) * d] = (
                at[:rmax, k:k + 1] * p[kk * rmax:(kk + 1) * rmax, :])
    acc = jnp.zeros((n, d), dtype=f32)
    for t in range(nt):
        rmax = (t + 1) * TK
        y = jnp.dot(a[:, :rmax], q_ref[:rmax, t * TK * d:(t + 1) * TK * d],
                    preferred_element_type=f32)
        for kk in range(TK):
            k = t * TK + kk
            acc = acc + a[:, k:k + 1] * y[:, kk * d:(kk + 1) * d]
    out_ref[0] = acc


def _mlp_body(xstd_ref, xpair_ref, w1_ref, b1_ref, g1_ref, be1_ref,
              w2_ref, b2_ref, g2_ref, be2_ref, out_ref):
    f32 = jnp.float32
    inv = 1.0 / math.sqrt(1.0 + 1e-5)
    xc = xstd_ref[:, :] + xpair_ref[:, :]
    h1 = jnp.maximum(
        jnp.dot(xc, w1_ref[:, :], preferred_element_type=f32) + b1_ref[:, :], 0.0)
    h1 = h1 * (inv * g1_ref[:, :]) + be1_ref[:, :]
    h2 = jnp.maximum(
        jnp.dot(h1, w2_ref[:, :], preferred_element_type=f32) + b2_ref[:, :], 0.0)
    out_ref[:, :] = h2 * (inv * g2_ref[:, :]) + be2_ref[:, :]


def kernel(A, X, eps, W1, b1, g1, be1, W2, b2, g2, be2, V1, c1, V2, c2):
    b, n = A.shape[0], A.shape[1]
    d_in, d_h = W1.shape
    fixed = lambda *zeros: (lambda i: zeros)
    xstd2 = _sc_xstd(A.reshape(b * n, n), X.reshape(b * n, d_in),
                     jnp.broadcast_to(eps, (SC_LANES,)))
    xpair = pl.pallas_call(
        _pairs_body,
        grid=(b,),
        in_specs=[
            pl.BlockSpec((1, n, n), lambda i: (i, 0, 0)),
            pl.BlockSpec((1, n, d_in), lambda i: (i, 0, 0)),
            pl.BlockSpec((d_in, d_h), fixed(0, 0)),
            pl.BlockSpec((1, d_h), fixed(0, 0)),
            pl.BlockSpec((d_h, d_in), fixed(0, 0)),
            pl.BlockSpec((1, d_in), fixed(0, 0)),
        ],
        out_specs=pl.BlockSpec((1, n, d_in), lambda i: (i, 0, 0)),
        out_shape=jax.ShapeDtypeStruct((b, n, d_in), jnp.float32),
        scratch_shapes=[
            pltpu.VMEM((n, n * d_in), jnp.float32),
            pltpu.VMEM((n, d_h), jnp.float32),
        ],
        compiler_params=pltpu.CompilerParams(
            dimension_semantics=("parallel",),
        ),
    )(A, X, V1, c1.reshape(1, d_h), V2, c2.reshape(1, d_in))
    out2 = pl.pallas_call(
        _mlp_body,
        out_shape=jax.ShapeDtypeStruct((b * n, d_h), jnp.float32),
    )(
        xstd2, xpair.reshape(b * n, d_in), W1, b1.reshape(1, d_h),
        jnp.tile(g1, b).reshape(b * n, 1), jnp.tile(be1, b).reshape(b * n, 1),
        W2, b2.reshape(1, d_h),
        jnp.tile(g2, b).reshape(b * n, 1), jnp.tile(be2, b).reshape(b * n, 1),
    )
    return out2.reshape(b, n, d_h)


# speedup vs baseline: 2.5476x; 2.5476x over previous
import math

import jax
import jax.numpy as jnp
from jax.experimental import pallas as pl
from jax.experimental.pallas import tpu as pltpu
from jax.experimental.pallas import tpu_sc as plsc

TK = 8
SC_LANES = 16
SC_SUBCORES = 16
SC_CORES = 2


def _sc_xstd_body(a_hbm, x_hbm, eps_hbm, o_hbm, a_loc, x_loc, eps_loc,
                  o_loc, idx_loc, sem):
    n = a_loc.shape[1]
    d = x_loc.shape[1]
    rows = a_loc.shape[0]
    nc = d // SC_LANES
    njc = n // SC_LANES
    core = jax.lax.axis_index("core")
    sub = jax.lax.axis_index("subcore")
    wid = core * SC_SUBCORES + sub
    r0 = wid * rows
    b = r0 // n
    i0 = r0 - b * n
    pltpu.async_copy(eps_hbm, eps_loc, sem).wait()
    pltpu.async_copy(a_hbm.at[pl.ds(r0, rows), :], a_loc, sem).wait()
    pltpu.async_copy(x_hbm.at[pl.ds(b * n, n), :], x_loc.at[pl.ds(0, n), :],
                     sem).wait()
    zero = jnp.zeros((SC_LANES,), jnp.float32)
    for c in range(nc):
        x_loc[n, pl.ds(c * SC_LANES, SC_LANES)] = zero
    s_vec = eps_loc[pl.ds(0, SC_LANES)] + 1.0
    sent = jnp.full((SC_LANES,), n, jnp.int32)

    @pl.loop(0, rows)
    def _row(li):
        for jc in range(njc + 1):
            idx_loc[pl.ds(jc * SC_LANES, SC_LANES)] = sent

        def compact(jc, off):
            av = a_loc[li, pl.ds(jc * SC_LANES, SC_LANES)]
            m = av != 0.0
            iv = jax.lax.iota(jnp.int32, SC_LANES) + jc * SC_LANES
            plsc.store_compressed(idx_loc.at[pl.ds(off, SC_LANES)], iv, mask=m)
            return off + plsc.all_reduce_population_count(m)[0]

        cnt = jax.lax.fori_loop(0, njc, compact, 0)

        acc0 = tuple(s_vec * x_loc[i0 + li, pl.ds(c * SC_LANES, SC_LANES)]
                     for c in range(nc))

        def edge_chunk(ec, acc):
            jv = idx_loc[pl.ds(ec * SC_LANES, SC_LANES)]
            for ll in range(SC_LANES):
                j = jv[ll]
                acc = tuple(acc[c] + x_loc[j, pl.ds(c * SC_LANES, SC_LANES)]
                            for c in range(nc))
            return acc

        nec = (cnt + SC_LANES - 1) // SC_LANES
        acc = jax.lax.fori_loop(0, nec, edge_chunk, acc0)
        for c in range(nc):
            o_loc[li, pl.ds(c * SC_LANES, SC_LANES)] = acc[c]

    pltpu.async_copy(o_loc, o_hbm.at[pl.ds(r0, rows), :], sem).wait()


def _sc_xstd(a2, x2, eps16):
    bn, n = x2.shape[0], x2.shape[1]
    rows = bn // (SC_CORES * SC_SUBCORES)
    mesh = plsc.VectorSubcoreMesh(core_axis_name="core",
                                  subcore_axis_name="subcore")
    kern = pl.kernel(
        _sc_xstd_body,
        out_type=jax.ShapeDtypeStruct((bn, n), jnp.float32),
        mesh=mesh,
        scratch_types=[
            pltpu.VMEM((rows, n), jnp.float32),
            pltpu.VMEM((n + 1, n), jnp.float32),
            pltpu.VMEM((SC_LANES,), jnp.float32),
            pltpu.VMEM((rows, n), jnp.float32),
            pltpu.VMEM((n + SC_LANES,), jnp.int32),
            pltpu.SemaphoreType.DMA,
        ],
        compiler_params=pltpu.CompilerParams(needs_layout_passes=False),
    )
    return kern(a2, x2, eps16)


def _pairs_body(a_ref, x_ref, v1_ref, c1_ref, v2_ref, c2_ref,
                out_ref, q_ref, g_ref):
    n = a_ref.shape[1]
    d = x_ref.shape[2]
    nt = n // TK
    f32 = jnp.float32
    a = a_ref[0]
    x = x_ref[0]
    g_ref[:, :] = jnp.dot(x, v1_ref[:, :], preferred_element_type=f32) \
        + 0.5 * c1_ref[:, :]
    G = g_ref[:, :]
    c2 = c2_ref[:, :]
    v2 = v2_ref[:, :]
    rowi = jax.lax.broadcasted_iota(jnp.int32, (n, n), 0)
    coli = jax.lax.broadcasted_iota(jnp.int32, (n, n), 1)
    at = jnp.where(rowi < coli, a, 0.0)
    for t in range(nt):
        rmax = (t + 1) * TK
        gk = G[t * TK:(t + 1) * TK, :]
        h = gk[:, None, :] + G[:rmax][None, :, :]
        h = jnp.maximum(h, 0.0).reshape(TK * rmax, -1)
        p = jnp.dot(h, v2, preferred_element_type=f32) + c2
        p = jnp.maximum(p, 0.0)
        for kk in range(TK):
            k = t * TK + kk
            q_ref[:rmax, k * d:(k + 1) * d] = (
                at[:rmax, k:k + 1] * p[kk * rmax:(kk + 1) * rmax, :])
    acc = jnp.zeros((n, d), dtype=f32)
    for t in range(nt):
        rmax = (t + 1) * TK
        y = jnp.dot(a[:, :rmax], q_ref[:rmax, t * TK * d:(t + 1) * TK * d],
                    preferred_element_type=f32)
        for kk in range(TK):
            k = t * TK + kk
            acc = acc + a[:, k:k + 1] * y[:, kk * d:(kk + 1) * d]
    out_ref[0] = acc


def _mlp_body(xstd_ref, xpair_ref, w1_ref, b1_ref, g1_ref, be1_ref,
              w2_ref, b2_ref, g2_ref, be2_ref, out_ref):
    f32 = jnp.float32
    inv = 1.0 / math.sqrt(1.0 + 1e-5)
    xc = xstd_ref[:, :] + xpair_ref[:, :]
    h1 = jnp.maximum(
        jnp.dot(xc, w1_ref[:, :], preferred_element_type=f32) + b1_ref[:, :], 0.0)
    h1 = h1 * (inv * g1_ref[:, :]) + be1_ref[:, :]
    h2 = jnp.maximum(
        jnp.dot(h1, w2_ref[:, :], preferred_element_type=f32) + b2_ref[:, :], 0.0)
    out_ref[:, :] = h2 * (inv * g2_ref[:, :]) + be2_ref[:, :]


def kernel(A, X, eps, W1, b1, g1, be1, W2, b2, g2, be2, V1, c1, V2, c2):
    b, n = A.shape[0], A.shape[1]
    d_in, d_h = W1.shape
    fixed = lambda *zeros: (lambda i: zeros)
    xstd2 = _sc_xstd(A.reshape(b * n, n), X.reshape(b * n, d_in),
                     jnp.broadcast_to(eps, (SC_LANES,)))
    xpair = pl.pallas_call(
        _pairs_body,
        grid=(b,),
        in_specs=[
            pl.BlockSpec((1, n, n), lambda i: (i, 0, 0)),
            pl.BlockSpec((1, n, d_in), lambda i: (i, 0, 0)),
            pl.BlockSpec((d_in, d_h), fixed(0, 0)),
            pl.BlockSpec((1, d_h), fixed(0, 0)),
            pl.BlockSpec((d_h, d_in), fixed(0, 0)),
            pl.BlockSpec((1, d_in), fixed(0, 0)),
        ],
        out_specs=pl.BlockSpec((1, n, d_in), lambda i: (i, 0, 0)),
        out_shape=jax.ShapeDtypeStruct((b, n, d_in), jnp.float32),
        scratch_shapes=[
            pltpu.VMEM((n, n * d_in), jnp.float32),
            pltpu.VMEM((n, d_h), jnp.float32),
        ],
        compiler_params=pltpu.CompilerParams(
            dimension_semantics=("parallel",),
        ),
    )(A, X, V1, c1.reshape(1, d_h), V2, c2.reshape(1, d_in))
    out2 = pl.pallas_call(
        _mlp_body,
        out_shape=jax.ShapeDtypeStruct((b * n, d_h), jnp.float32),
    )(
        xstd2, xpair.reshape(b * n, d_in), W1, b1.reshape(1, d_h),
        jnp.tile(g1, b).reshape(b * n, 1), jnp.tile(be1, b).reshape(b * n, 1),
        W2, b2.reshape(1, d_h),
        jnp.tile(g2, b).reshape(b * n, 1), jnp.tile(be2, b).reshape(b * n, 1),
    )
    return out2.reshape(b, n, d_h)


# speedup vs baseline: 4.3636x; 1.7129x over previous
import math

import jax
import jax.numpy as jnp
from jax.experimental import pallas as pl
from jax.experimental.pallas import tpu as pltpu

TK = 8


def _body(eps_ref, a_ref, x_ref, w1_ref, b1_ref, g1_ref, be1_ref,
          w2_ref, b2_ref, g2_ref, be2_ref, v1_ref, c1_ref, v2_ref, c2_ref,
          out_ref, q_ref, g_ref):
    n = a_ref.shape[1]
    d = x_ref.shape[2]
    nt = n // TK
    f32 = jnp.float32
    a = a_ref[0]
    x = x_ref[0]
    g_ref[:, :] = (jnp.dot(x, v1_ref[:, :], preferred_element_type=f32)
                   + 0.5 * c1_ref[:, :]).astype(jnp.bfloat16)
    G = g_ref[:, :]
    c2 = c2_ref[:, :]
    v2 = v2_ref[:, :].astype(jnp.bfloat16)
    rowi = jax.lax.broadcasted_iota(jnp.int32, (n, n), 0)
    coli = jax.lax.broadcasted_iota(jnp.int32, (n, n), 1)
    at = jnp.where(rowi < coli, a, 0.0)
    for t in range(nt):
        rmax = (t + 1) * TK
        gk = G[t * TK:(t + 1) * TK, :]
        h = gk[:, None, :] + G[:rmax][None, :, :]
        h = jnp.maximum(h, jnp.bfloat16(0.0)).reshape(TK * rmax, -1)
        p = jnp.dot(h, v2, preferred_element_type=f32) + c2
        p = jnp.maximum(p, 0.0)
        for kk in range(TK):
            k = t * TK + kk
            q_ref[:rmax, k * d:(k + 1) * d] = (
                at[:rmax, k:k + 1] * p[kk * rmax:(kk + 1) * rmax, :])
    acc = jnp.zeros((n, d), dtype=f32)
    for t in range(nt):
        rmax = (t + 1) * TK
        y = jnp.dot(a[:, :rmax], q_ref[:rmax, t * TK * d:(t + 1) * TK * d],
                    preferred_element_type=f32)
        for kk in range(TK):
            k = t * TK + kk
            acc = acc + a[:, k:k + 1] * y[:, kk * d:(kk + 1) * d]
    xc = (1.0 + eps_ref[0, 0]) * x + jnp.dot(a, x, preferred_element_type=f32) \
        + acc
    inv = 1.0 / math.sqrt(1.0 + 1e-5)
    h1 = jnp.maximum(
        jnp.dot(xc, w1_ref[:, :], preferred_element_type=f32) + b1_ref[:, :], 0.0)
    h1 = h1 * (inv * g1_ref[:, :]) + be1_ref[:, :]
    h2 = jnp.maximum(
        jnp.dot(h1, w2_ref[:, :], preferred_element_type=f32) + b2_ref[:, :], 0.0)
    out_ref[0] = h2 * (inv * g2_ref[:, :]) + be2_ref[:, :]


def kernel(A, X, eps, W1, b1, g1, be1, W2, b2, g2, be2, V1, c1, V2, c2):
    b, n = A.shape[0], A.shape[1]
    d_in, d_h = W1.shape
    fixed = lambda *zeros: (lambda i: zeros)
    out = pl.pallas_call(
        _body,
        grid=(b,),
        in_specs=[
            pl.BlockSpec((1, 1), fixed(0, 0), memory_space=pltpu.SMEM),
            pl.BlockSpec((1, n, n), lambda i: (i, 0, 0)),
            pl.BlockSpec((1, n, d_in), lambda i: (i, 0, 0)),
            pl.BlockSpec((d_in, d_h), fixed(0, 0)),
            pl.BlockSpec((1, d_h), fixed(0, 0)),
            pl.BlockSpec((n, 1), fixed(0, 0)),
            pl.BlockSpec((n, 1), fixed(0, 0)),
            pl.BlockSpec((d_h, d_h), fixed(0, 0)),
            pl.BlockSpec((1, d_h), fixed(0, 0)),
            pl.BlockSpec((n, 1), fixed(0, 0)),
            pl.BlockSpec((n, 1), fixed(0, 0)),
            pl.BlockSpec((d_in, d_h), fixed(0, 0)),
            pl.BlockSpec((1, d_h), fixed(0, 0)),
            pl.BlockSpec((d_h, d_in), fixed(0, 0)),
            pl.BlockSpec((1, d_in), fixed(0, 0)),
        ],
        out_specs=pl.BlockSpec((1, n, d_h), lambda i: (i, 0, 0)),
        out_shape=jax.ShapeDtypeStruct((b, n, d_h), jnp.float32),
        scratch_shapes=[
            pltpu.VMEM((n, n * d_in), jnp.float32),
            pltpu.VMEM((n, d_h), jnp.bfloat16),
        ],
        compiler_params=pltpu.CompilerParams(
            dimension_semantics=("parallel",),
        ),
    )(
        eps.reshape(1, 1), A, X, W1, b1.reshape(1, d_h), g1.reshape(n, 1),
        be1.reshape(n, 1), W2, b2.reshape(1, d_h), g2.reshape(n, 1),
        be2.reshape(n, 1), V1, c1.reshape(1, d_h), V2, c2.reshape(1, d_in),
    )
    return out
